# baseline (speedup 1.0000x reference)
import functools

import jax
import jax.numpy as jnp
from jax import lax
from jax.experimental import pallas as pl
from jax.experimental.pallas import tpu as pltpu
from jax.experimental.pallas import tpu_sc as plsc

N = 10000
E = 320000
D = 128
DE = 16
DOUT = 128

NC = 2
NS = 16
L = 16
NW = NC * NS
EPW = E // NW
C = 64
NCH = 156
PAIRS = NCH // 2
TAILE = EPW - NCH * C
TBASE = NCH * C
NBLK = N // C
AGG_TAIL0 = NBLK * C
AGG_TAILR = N - AGG_TAIL0


def _proj_body(x_ref, w1_ref, w2_ref, o1_ref, o2_ref):
    xb = x_ref[...]
    o1_ref[...] = jnp.dot(xb, w1_ref[...], preferred_element_type=jnp.float32)
    o2_ref[...] = jnp.dot(xb, w2_ref[...], preferred_element_type=jnp.float32)


def _proj(x, w1, w2):
    bn = 1000
    grid = N // bn
    return pl.pallas_call(
        _proj_body,
        grid=(grid,),
        in_specs=[
            pl.BlockSpec((bn, D), lambda i: (i, 0)),
            pl.BlockSpec((D, D), lambda i: (0, 0)),
            pl.BlockSpec((D, D), lambda i: (0, 0)),
        ],
        out_specs=[
            pl.BlockSpec((bn, D), lambda i: (i, 0)),
            pl.BlockSpec((bn, D), lambda i: (i, 0)),
        ],
        out_shape=[
            jax.ShapeDtypeStruct((N, D), jnp.float32),
            jax.ShapeDtypeStruct((N, D), jnp.float32),
        ],
    )(x, w1, w2)


def _patt_body(a_ref, w_ref, b_ref, o_ref):
    o_ref[...] = jnp.dot(a_ref[...], w_ref[...],
                         preferred_element_type=jnp.float32) + b_ref[...]


def _patt(edge_attr, w3, b_e):
    be = 4000
    grid = E // be
    return pl.pallas_call(
        _patt_body,
        grid=(grid,),
        in_specs=[
            pl.BlockSpec((be, DE), lambda i: (i, 0)),
            pl.BlockSpec((DE, DOUT), lambda i: (0, 0)),
            pl.BlockSpec((1, DOUT), lambda i: (0, 0)),
        ],
        out_specs=pl.BlockSpec((be, DOUT), lambda i: (i, 0)),
        out_shape=jax.ShapeDtypeStruct((E, DOUT), jnp.float32),
    )(edge_attr, w3, b_e)


def _node_body(x_ref, a_ref, w1_ref, w2_ref, b_ref, o_ref):
    acc = jnp.dot(x_ref[...], w1_ref[...], preferred_element_type=jnp.float32)
    acc += jnp.dot(a_ref[0] + a_ref[1], w2_ref[...],
                   preferred_element_type=jnp.float32)
    o_ref[...] = jnp.maximum(acc + b_ref[...], 0.0)


def _node(x, aggs, wn1, wn2, b_n):
    bn = 1000
    grid = N // bn
    return pl.pallas_call(
        _node_body,
        grid=(grid,),
        in_specs=[
            pl.BlockSpec((bn, D), lambda i: (i, 0)),
            pl.BlockSpec((NC, bn, DOUT), lambda i: (0, i, 0)),
            pl.BlockSpec((D, D), lambda i: (0, 0)),
            pl.BlockSpec((DOUT, D), lambda i: (0, 0)),
            pl.BlockSpec((1, D), lambda i: (0, 0)),
        ],
        out_specs=pl.BlockSpec((bn, D), lambda i: (i, 0)),
        out_shape=jax.ShapeDtypeStruct((N, D), jnp.float32),
    )(x, aggs, wn1, wn2, b_n)


_sc_mesh = plsc.VectorSubcoreMesh(core_axis_name="c", subcore_axis_name="s")


@functools.partial(
    pl.kernel,
    out_type=(
        jax.ShapeDtypeStruct((E, DOUT), jnp.float32),
        jax.ShapeDtypeStruct((NC, N, DOUT), jnp.float32),
    ),
    mesh=_sc_mesh,
    compiler_params=pltpu.CompilerParams(use_tc_tiling_on_sc=True),
    scratch_types=[
        pltpu.VMEM((C,), jnp.int32),
        pltpu.VMEM((C,), jnp.int32),
        pltpu.VMEM((C,), jnp.int32),
        pltpu.VMEM((C,), jnp.int32),
        pltpu.VMEM((C,), jnp.int32),
        pltpu.VMEM((C,), jnp.int32),
        pltpu.VMEM((TAILE,), jnp.int32),
        pltpu.VMEM((TAILE,), jnp.int32),
        pltpu.VMEM((C, DOUT), jnp.float32),
        pltpu.VMEM((C, DOUT), jnp.float32),
        pltpu.VMEM((C, DOUT), jnp.float32),
        pltpu.VMEM((C, DOUT), jnp.float32),
        pltpu.VMEM((C, DOUT), jnp.float32),
        pltpu.VMEM((C, DOUT), jnp.float32),
        pltpu.VMEM_SHARED((N, DOUT), jnp.float32),
        pltpu.SemaphoreType.DMA,
        pltpu.SemaphoreType.DMA,
        pltpu.SemaphoreType.DMA,
        pltpu.SemaphoreType.DMA,
        pltpu.SemaphoreType.DMA,
        pltpu.SemaphoreType.DMA,
        pltpu.SemaphoreType.DMA,
        pltpu.SemaphoreType.DMA,
        pltpu.SemaphoreType.DMA,
        pltpu.SemaphoreType.DMA,
        pltpu.SemaphoreType.DMA,
        pltpu.SemaphoreType.DMA,
    ],
)
def _edge_kernel(src_hbm, dst_hbm, psrc_hbm, pdst_hbm, patt_hbm,
                 eout_hbm, agg_hbm,
                 idx_s0, idx_d0, idx_s1, idx_d1, sidx0, sidx1,
                 idx_st, idx_dt,
                 a0, b0, c0, a1, b1, c1, agg_sh,
                 ga_sem0, gb_sem0, pc_sem0, ga_sem1, gb_sem1, pc_sem1,
                 eo_sem0, sc_sem0, eo_sem1, sc_sem1, ix_sem0, ix_sem1):
    cid = lax.axis_index("c")
    sid = lax.axis_index("s")
    wid = sid * NC + cid
    base_w = wid * EPW

    sets = (
        dict(idx_s=idx_s0, idx_d=idx_d0, a=a0, b=b0, c=c0, sidx=sidx0,
             ga=ga_sem0, gb=gb_sem0, pc=pc_sem0, eo=eo_sem0, sc=sc_sem0,
             ix=ix_sem0),
        dict(idx_s=idx_s1, idx_d=idx_d1, a=a1, b=b1, c=c1, sidx=sidx1,
             ga=ga_sem1, gb=gb_sem1, pc=pc_sem1, eo=eo_sem1, sc=sc_sem1,
             ix=ix_sem1),
    )

    def idx_descs(base, s):
        t = sets[s]
        return (pltpu.make_async_copy(src_hbm.at[pl.ds(base, C)],
                                      t["idx_s"], t["ix"]),
                pltpu.make_async_copy(dst_hbm.at[pl.ds(base, C)],
                                      t["idx_d"], t["ix"]))

    def in_descs(base, s):
        t = sets[s]
        return (pltpu.make_async_copy(psrc_hbm.at[t["idx_s"]],
                                      t["a"], t["ga"]),
                pltpu.make_async_copy(pdst_hbm.at[t["idx_d"]],
                                      t["b"], t["gb"]),
                pltpu.make_async_copy(patt_hbm.at[pl.ds(base, C)],
                                      t["c"], t["pc"]))

    def out_descs(base, s):
        t = sets[s]
        return (pltpu.make_async_copy(t["a"],
                                      eout_hbm.at[pl.ds(base, C)],
                                      t["eo"]),
                pltpu.make_async_copy(t["a"],
                                      agg_sh.at[t["sidx"]], t["sc"]))

    def snap_idx(s):
        t = sets[s]
        for g in range(C // L):
            sl = pl.ds(g * L, L)
            t["sidx"][sl] = t["idx_d"][sl]

    def fire_out(base, s):
        d = out_descs(base, s)
        d[0].start()
        d[1].start(add=True)

    def _compute(a, b, c, nrows):
        def _row(i, rcarry):
            for g in range(DOUT // L):
                sl = pl.ds(g * L, L)
                a[i, sl] = jnp.maximum(a[i, sl] + b[i, sl] + c[i, sl], 0.0)
            return rcarry

        lax.fori_loop(0, nrows, _row, 0)

    def _zrow(i, carry):
        for g in range(DOUT // L):
            a0[i, pl.ds(g * L, L)] = jnp.zeros((L,), jnp.float32)
        return carry

    lax.fori_loop(0, C, _zrow, 0)
    nblk_me = jnp.where(sid < NBLK - 9 * NS, 10, 9)

    def _zblk(k, carry):
        pltpu.sync_copy(a0, agg_sh.at[pl.ds((sid + NS * k) * C, C)])
        return carry

    lax.fori_loop(0, nblk_me, _zblk, 0)

    @pl.when(sid == NS - 1)
    def _zero_tail():
        pltpu.sync_copy(a0.at[pl.ds(0, AGG_TAILR)],
                        agg_sh.at[pl.ds(AGG_TAIL0, AGG_TAILR)])

    plsc.subcore_barrier()

    pltpu.sync_copy(src_hbm.at[pl.ds(base_w, C)], idx_s0)
    pltpu.sync_copy(dst_hbm.at[pl.ds(base_w, C)], idx_d0)
    for d in in_descs(base_w, 0):
        d.start()
    for d in idx_descs(base_w + C, 1):
        d.start()

    def _pair(p, carry):
        base_a = base_w + 2 * p * C

        @pl.when(p > 0)
        def _():
            for d in out_descs(base_a - C, 1):
                d.wait()

        for d in idx_descs(base_a + C, 1):
            d.wait()
        for d in in_descs(base_a + C, 1):
            d.start()
        for d in in_descs(base_a, 0):
            d.wait()
        snap_idx(0)

        @pl.when(p < PAIRS - 1)
        def _():
            for d in idx_descs(base_a + 2 * C, 0):
                d.start()

        _compute(a0, b0, c0, C)
        fire_out(base_a, 0)

        base_b = base_a + C
        for d in out_descs(base_a, 0):
            d.wait()

        @pl.when(p < PAIRS - 1)
        def _():
            for d in idx_descs(base_b + C, 0):
                d.wait()
            for d in in_descs(base_b + C, 0):
                d.start()

        for d in in_descs(base_b, 1):
            d.wait()
        snap_idx(1)

        @pl.when(p < PAIRS - 1)
        def _():
            for d in idx_descs(base_b + 2 * C, 1):
                d.start()

        _compute(a1, b1, c1, C)
        fire_out(base_b, 1)
        return carry

    lax.fori_loop(0, PAIRS, _pair, 0)
    for d in out_descs(base_w + (NCH - 1) * C, 1):
        d.wait()

    tb = base_w + TBASE
    pltpu.sync_copy(src_hbm.at[pl.ds(tb, TAILE)], idx_st)
    pltpu.sync_copy(dst_hbm.at[pl.ds(tb, TAILE)], idx_dt)
    ta = a0.at[pl.ds(0, TAILE)]
    cp1 = pltpu.async_copy(psrc_hbm.at[idx_st], ta, ga_sem0)
    cp2 = pltpu.async_copy(pdst_hbm.at[idx_dt], b0.at[pl.ds(0, TAILE)],
                           gb_sem0)
    cp3 = pltpu.async_copy(patt_hbm.at[pl.ds(tb, TAILE)],
                           c0.at[pl.ds(0, TAILE)], pc_sem0)
    cp1.wait()
    cp2.wait()
    cp3.wait()
    _compute(a0, b0, c0, TAILE)
    pltpu.sync_copy(ta, eout_hbm.at[pl.ds(tb, TAILE)])
    pltpu.sync_copy(ta, agg_sh.at[idx_dt], add=True)

    plsc.subcore_barrier()

    def _dblk(k, carry):
        off = (sid + NS * k) * C
        pltpu.sync_copy(agg_sh.at[pl.ds(off, C)], a0)
        pltpu.sync_copy(a0, agg_hbm.at[cid, pl.ds(off, C)])
        return carry

    lax.fori_loop(0, nblk_me, _dblk, 0)

    @pl.when(sid == NS - 1)
    def _drain_tail():
        pltpu.sync_copy(agg_sh.at[pl.ds(AGG_TAIL0, AGG_TAILR)],
                        a0.at[pl.ds(0, AGG_TAILR)])
        pltpu.sync_copy(a0.at[pl.ds(0, AGG_TAILR)],
                        agg_hbm.at[cid, pl.ds(AGG_TAIL0, AGG_TAILR)])


@jax.jit
def kernel(x, edge_index, edge_attr, W_e, b_e, W_n, b_n):
    src = edge_index[0]
    dst = edge_index[1]
    psrc, pdst = _proj(x, W_e[:D], W_e[D:2 * D])
    patt = _patt(edge_attr, W_e[2 * D:], b_e.reshape(1, DOUT))
    edge_out, aggs = _edge_kernel(src, dst, psrc, pdst, patt)
    x_out = _node(x, aggs, W_n[:D], W_n[D:], b_n.reshape(1, D))
    return (x_out, edge_out)


# speedup vs baseline: 1.3005x; 1.3005x over previous
import functools

import jax
import jax.numpy as jnp
from jax import lax
from jax.experimental import pallas as pl
from jax.experimental.pallas import tpu as pltpu
from jax.experimental.pallas import tpu_sc as plsc

N = 10000
E = 320000
D = 128
DE = 16
DOUT = 128

NC = 2
NS = 16
L = 16
NW = NC * NS
EPW = E // NW
C = 64
NCH = 156
PAIRS = NCH // 2
TAILE = EPW - NCH * C
TBASE = NCH * C
NBLK = N // C
AGG_TAIL0 = NBLK * C
AGG_TAILR = N - AGG_TAIL0


def _proj_body(x_ref, w1_ref, w2_ref, o1_ref, o2_ref):
    xb = x_ref[...]
    o1_ref[...] = jnp.dot(xb, w1_ref[...], preferred_element_type=jnp.float32)
    o2_ref[...] = jnp.dot(xb, w2_ref[...], preferred_element_type=jnp.float32)


def _proj(x, w1, w2):
    bn = 1000
    grid = N // bn
    return pl.pallas_call(
        _proj_body,
        grid=(grid,),
        in_specs=[
            pl.BlockSpec((bn, D), lambda i: (i, 0)),
            pl.BlockSpec((D, D), lambda i: (0, 0)),
            pl.BlockSpec((D, D), lambda i: (0, 0)),
        ],
        out_specs=[
            pl.BlockSpec((bn, D), lambda i: (i, 0)),
            pl.BlockSpec((bn, D), lambda i: (i, 0)),
        ],
        out_shape=[
            jax.ShapeDtypeStruct((N, D), jnp.float32),
            jax.ShapeDtypeStruct((N, D), jnp.float32),
        ],
    )(x, w1, w2)


def _patt_body(at_ref, w_ref, b_ref, o_ref):
    o_ref[...] = lax.dot_general(
        at_ref[...], w_ref[...],
        dimension_numbers=(((0,), (0,)), ((), ())),
        preferred_element_type=jnp.float32) + b_ref[...]


def _patt(edge_attr_t, w3, b_e):
    be = 6400
    grid = E // be
    return pl.pallas_call(
        _patt_body,
        grid=(grid,),
        in_specs=[
            pl.BlockSpec((DE, be), lambda i: (0, i)),
            pl.BlockSpec((DE, DOUT), lambda i: (0, 0)),
            pl.BlockSpec((1, DOUT), lambda i: (0, 0)),
        ],
        out_specs=pl.BlockSpec((be, DOUT), lambda i: (i, 0)),
        out_shape=jax.ShapeDtypeStruct((E, DOUT), jnp.float32),
    )(edge_attr_t, w3, b_e)


def _node_body(x_ref, a_ref, w1_ref, w2_ref, b_ref, o_ref):
    acc = jnp.dot(x_ref[...], w1_ref[...], preferred_element_type=jnp.float32)
    acc += jnp.dot(a_ref[0] + a_ref[1], w2_ref[...],
                   preferred_element_type=jnp.float32)
    o_ref[...] = jnp.maximum(acc + b_ref[...], 0.0)


def _node(x, aggs, wn1, wn2, b_n):
    bn = 1000
    grid = N // bn
    return pl.pallas_call(
        _node_body,
        grid=(grid,),
        in_specs=[
            pl.BlockSpec((bn, D), lambda i: (i, 0)),
            pl.BlockSpec((NC, bn, DOUT), lambda i: (0, i, 0)),
            pl.BlockSpec((D, D), lambda i: (0, 0)),
            pl.BlockSpec((DOUT, D), lambda i: (0, 0)),
            pl.BlockSpec((1, D), lambda i: (0, 0)),
        ],
        out_specs=pl.BlockSpec((bn, D), lambda i: (i, 0)),
        out_shape=jax.ShapeDtypeStruct((N, D), jnp.float32),
    )(x, aggs, wn1, wn2, b_n)


_sc_mesh = plsc.VectorSubcoreMesh(core_axis_name="c", subcore_axis_name="s")


@functools.partial(
    pl.kernel,
    out_type=(
        jax.ShapeDtypeStruct((E, DOUT), jnp.float32),
        jax.ShapeDtypeStruct((NC, N, DOUT), jnp.float32),
    ),
    mesh=_sc_mesh,
    compiler_params=pltpu.CompilerParams(use_tc_tiling_on_sc=True),
    scratch_types=[
        pltpu.VMEM((C,), jnp.int32),
        pltpu.VMEM((C,), jnp.int32),
        pltpu.VMEM((C,), jnp.int32),
        pltpu.VMEM((C,), jnp.int32),
        pltpu.VMEM((C,), jnp.int32),
        pltpu.VMEM((C,), jnp.int32),
        pltpu.VMEM((TAILE,), jnp.int32),
        pltpu.VMEM((TAILE,), jnp.int32),
        pltpu.VMEM((C, DOUT), jnp.float32),
        pltpu.VMEM((C, DOUT), jnp.float32),
        pltpu.VMEM((C, DOUT), jnp.float32),
        pltpu.VMEM((C, DOUT), jnp.float32),
        pltpu.VMEM((C, DOUT), jnp.float32),
        pltpu.VMEM((C, DOUT), jnp.float32),
        pltpu.VMEM_SHARED((N, DOUT), jnp.float32),
        pltpu.SemaphoreType.DMA,
        pltpu.SemaphoreType.DMA,
        pltpu.SemaphoreType.DMA,
        pltpu.SemaphoreType.DMA,
        pltpu.SemaphoreType.DMA,
        pltpu.SemaphoreType.DMA,
        pltpu.SemaphoreType.DMA,
        pltpu.SemaphoreType.DMA,
        pltpu.SemaphoreType.DMA,
        pltpu.SemaphoreType.DMA,
        pltpu.SemaphoreType.DMA,
        pltpu.SemaphoreType.DMA,
    ],
)
def _edge_kernel(src_hbm, dst_hbm, psrc_hbm, pdst_hbm, patt_hbm,
                 eout_hbm, agg_hbm,
                 idx_s0, idx_d0, idx_s1, idx_d1, sidx0, sidx1,
                 idx_st, idx_dt,
                 a0, b0, c0, a1, b1, c1, agg_sh,
                 ga_sem0, gb_sem0, pc_sem0, ga_sem1, gb_sem1, pc_sem1,
                 eo_sem0, sc_sem0, eo_sem1, sc_sem1, ix_sem0, ix_sem1):
    cid = lax.axis_index("c")
    sid = lax.axis_index("s")
    wid = sid * NC + cid
    base_w = wid * EPW

    sets = (
        dict(idx_s=idx_s0, idx_d=idx_d0, a=a0, b=b0, c=c0, sidx=sidx0,
             ga=ga_sem0, gb=gb_sem0, pc=pc_sem0, eo=eo_sem0, sc=sc_sem0,
             ix=ix_sem0),
        dict(idx_s=idx_s1, idx_d=idx_d1, a=a1, b=b1, c=c1, sidx=sidx1,
             ga=ga_sem1, gb=gb_sem1, pc=pc_sem1, eo=eo_sem1, sc=sc_sem1,
             ix=ix_sem1),
    )

    def idx_descs(base, s):
        t = sets[s]
        return (pltpu.make_async_copy(src_hbm.at[pl.ds(base, C)],
                                      t["idx_s"], t["ix"]),
                pltpu.make_async_copy(dst_hbm.at[pl.ds(base, C)],
                                      t["idx_d"], t["ix"]))

    def in_descs(base, s):
        t = sets[s]
        return (pltpu.make_async_copy(psrc_hbm.at[t["idx_s"]],
                                      t["a"], t["ga"]),
                pltpu.make_async_copy(pdst_hbm.at[t["idx_d"]],
                                      t["b"], t["gb"]),
                pltpu.make_async_copy(patt_hbm.at[pl.ds(base, C)],
                                      t["c"], t["pc"]))

    def out_descs(base, s):
        t = sets[s]
        return (pltpu.make_async_copy(t["a"],
                                      eout_hbm.at[pl.ds(base, C)],
                                      t["eo"]),
                pltpu.make_async_copy(t["a"],
                                      agg_sh.at[t["sidx"]], t["sc"]))

    def snap_idx(s):
        t = sets[s]
        for g in range(C // L):
            sl = pl.ds(g * L, L)
            t["sidx"][sl] = t["idx_d"][sl]

    def fire_out(base, s):
        d = out_descs(base, s)
        d[0].start()
        d[1].start(add=True)

    def _compute(a, b, c, nrows):
        def _row(i, rcarry):
            for g in range(DOUT // L):
                sl = pl.ds(g * L, L)
                a[i, sl] = jnp.maximum(a[i, sl] + b[i, sl] + c[i, sl], 0.0)
            return rcarry

        lax.fori_loop(0, nrows, _row, 0)

    def _zrow(i, carry):
        for g in range(DOUT // L):
            a0[i, pl.ds(g * L, L)] = jnp.zeros((L,), jnp.float32)
        return carry

    lax.fori_loop(0, C, _zrow, 0)
    nblk_me = jnp.where(sid < NBLK - 9 * NS, 10, 9)

    def _zblk(k, carry):
        pltpu.sync_copy(a0, agg_sh.at[pl.ds((sid + NS * k) * C, C)])
        return carry

    lax.fori_loop(0, nblk_me, _zblk, 0)

    @pl.when(sid == NS - 1)
    def _zero_tail():
        pltpu.sync_copy(a0.at[pl.ds(0, AGG_TAILR)],
                        agg_sh.at[pl.ds(AGG_TAIL0, AGG_TAILR)])

    plsc.subcore_barrier()

    pltpu.sync_copy(src_hbm.at[pl.ds(base_w, C)], idx_s0)
    pltpu.sync_copy(dst_hbm.at[pl.ds(base_w, C)], idx_d0)
    for d in in_descs(base_w, 0):
        d.start()
    for d in idx_descs(base_w + C, 1):
        d.start()

    def _pair(p, carry):
        base_a = base_w + 2 * p * C

        @pl.when(p > 0)
        def _():
            for d in out_descs(base_a - C, 1):
                d.wait()

        for d in idx_descs(base_a + C, 1):
            d.wait()
        for d in in_descs(base_a + C, 1):
            d.start()
        for d in in_descs(base_a, 0):
            d.wait()
        snap_idx(0)

        @pl.when(p < PAIRS - 1)
        def _():
            for d in idx_descs(base_a + 2 * C, 0):
                d.start()

        _compute(a0, b0, c0, C)
        fire_out(base_a, 0)

        base_b = base_a + C
        for d in out_descs(base_a, 0):
            d.wait()

        @pl.when(p < PAIRS - 1)
        def _():
            for d in idx_descs(base_b + C, 0):
                d.wait()
            for d in in_descs(base_b + C, 0):
                d.start()

        for d in in_descs(base_b, 1):
            d.wait()
        snap_idx(1)

        @pl.when(p < PAIRS - 1)
        def _():
            for d in idx_descs(base_b + 2 * C, 1):
                d.start()

        _compute(a1, b1, c1, C)
        fire_out(base_b, 1)
        return carry

    lax.fori_loop(0, PAIRS, _pair, 0)
    for d in out_descs(base_w + (NCH - 1) * C, 1):
        d.wait()

    tb = base_w + TBASE
    pltpu.sync_copy(src_hbm.at[pl.ds(tb, TAILE)], idx_st)
    pltpu.sync_copy(dst_hbm.at[pl.ds(tb, TAILE)], idx_dt)
    ta = a0.at[pl.ds(0, TAILE)]
    cp1 = pltpu.async_copy(psrc_hbm.at[idx_st], ta, ga_sem0)
    cp2 = pltpu.async_copy(pdst_hbm.at[idx_dt], b0.at[pl.ds(0, TAILE)],
                           gb_sem0)
    cp3 = pltpu.async_copy(patt_hbm.at[pl.ds(tb, TAILE)],
                           c0.at[pl.ds(0, TAILE)], pc_sem0)
    cp1.wait()
    cp2.wait()
    cp3.wait()
    _compute(a0, b0, c0, TAILE)
    pltpu.sync_copy(ta, eout_hbm.at[pl.ds(tb, TAILE)])
    pltpu.sync_copy(ta, agg_sh.at[idx_dt], add=True)

    plsc.subcore_barrier()

    def _dblk(k, carry):
        off = (sid + NS * k) * C
        pltpu.sync_copy(agg_sh.at[pl.ds(off, C)], a0)
        pltpu.sync_copy(a0, agg_hbm.at[cid, pl.ds(off, C)])
        return carry

    lax.fori_loop(0, nblk_me, _dblk, 0)

    @pl.when(sid == NS - 1)
    def _drain_tail():
        pltpu.sync_copy(agg_sh.at[pl.ds(AGG_TAIL0, AGG_TAILR)],
                        a0.at[pl.ds(0, AGG_TAILR)])
        pltpu.sync_copy(a0.at[pl.ds(0, AGG_TAILR)],
                        agg_hbm.at[cid, pl.ds(AGG_TAIL0, AGG_TAILR)])


@jax.jit
def kernel(x, edge_index, edge_attr, W_e, b_e, W_n, b_n):
    src = edge_index[0]
    dst = edge_index[1]
    psrc, pdst = _proj(x, W_e[:D], W_e[D:2 * D])
    patt = _patt(edge_attr.T, W_e[2 * D:], b_e.reshape(1, DOUT))
    edge_out, aggs = _edge_kernel(src, dst, psrc, pdst, patt)
    x_out = _node(x, aggs, W_n[:D], W_n[D:], b_n.reshape(1, D))
    return (x_out, edge_out)


# speedup vs baseline: 1.3089x; 1.0065x over previous
import functools

import jax
import jax.numpy as jnp
from jax import lax
from jax.experimental import pallas as pl
from jax.experimental.pallas import tpu as pltpu
from jax.experimental.pallas import tpu_sc as plsc

N = 10000
E = 320000
D = 128
DE = 16
DOUT = 128

NC = 2
NS = 16
L = 16
NW = NC * NS
EPW = E // NW
C = 64
NCH = 156
PAIRS = NCH // 2
TAILE = EPW - NCH * C
TBASE = NCH * C
NBLK = N // C
AGG_TAIL0 = NBLK * C
AGG_TAILR = N - AGG_TAIL0


def _proj_body(x_ref, w1_ref, w2_ref, o1_ref, o2_ref):
    xb = x_ref[...]
    o1_ref[...] = jnp.dot(xb, w1_ref[...], preferred_element_type=jnp.float32)
    o2_ref[...] = jnp.dot(xb, w2_ref[...], preferred_element_type=jnp.float32)


def _proj(x, w1, w2):
    bn = 2000
    grid = N // bn
    return pl.pallas_call(
        _proj_body,
        grid=(grid,),
        in_specs=[
            pl.BlockSpec((bn, D), lambda i: (i, 0)),
            pl.BlockSpec((D, D), lambda i: (0, 0)),
            pl.BlockSpec((D, D), lambda i: (0, 0)),
        ],
        out_specs=[
            pl.BlockSpec((bn, D), lambda i: (i, 0)),
            pl.BlockSpec((bn, D), lambda i: (i, 0)),
        ],
        out_shape=[
            jax.ShapeDtypeStruct((N, D), jnp.float32),
            jax.ShapeDtypeStruct((N, D), jnp.float32),
        ],
    )(x, w1, w2)


def _patt_body(at_ref, w_ref, b_ref, o_ref):
    o_ref[...] = lax.dot_general(
        at_ref[...], w_ref[...],
        dimension_numbers=(((0,), (0,)), ((), ())),
        preferred_element_type=jnp.float32) + b_ref[...]


def _patt(edge_attr_t, w3, b_e):
    be = 6400
    grid = E // be
    return pl.pallas_call(
        _patt_body,
        grid=(grid,),
        in_specs=[
            pl.BlockSpec((DE, be), lambda i: (0, i)),
            pl.BlockSpec((DE, DOUT), lambda i: (0, 0)),
            pl.BlockSpec((1, DOUT), lambda i: (0, 0)),
        ],
        out_specs=pl.BlockSpec((be, DOUT), lambda i: (i, 0)),
        out_shape=jax.ShapeDtypeStruct((E, DOUT), jnp.float32),
    )(edge_attr_t, w3, b_e)


def _node_body(x_ref, a_ref, w1_ref, w2_ref, b_ref, o_ref):
    acc = jnp.dot(x_ref[...], w1_ref[...], preferred_element_type=jnp.float32)
    acc += jnp.dot(a_ref[0] + a_ref[1], w2_ref[...],
                   preferred_element_type=jnp.float32)
    o_ref[...] = jnp.maximum(acc + b_ref[...], 0.0)


def _node(x, aggs, wn1, wn2, b_n):
    bn = 1000
    grid = N // bn
    return pl.pallas_call(
        _node_body,
        grid=(grid,),
        in_specs=[
            pl.BlockSpec((bn, D), lambda i: (i, 0)),
            pl.BlockSpec((NC, bn, DOUT), lambda i: (0, i, 0)),
            pl.BlockSpec((D, D), lambda i: (0, 0)),
            pl.BlockSpec((DOUT, D), lambda i: (0, 0)),
            pl.BlockSpec((1, D), lambda i: (0, 0)),
        ],
        out_specs=pl.BlockSpec((bn, D), lambda i: (i, 0)),
        out_shape=jax.ShapeDtypeStruct((N, D), jnp.float32),
    )(x, aggs, wn1, wn2, b_n)


_sc_mesh = plsc.VectorSubcoreMesh(core_axis_name="c", subcore_axis_name="s")


@functools.partial(
    pl.kernel,
    out_type=(
        jax.ShapeDtypeStruct((E, DOUT), jnp.float32),
        jax.ShapeDtypeStruct((NC, N, DOUT), jnp.float32),
    ),
    mesh=_sc_mesh,
    scratch_types=[
        pltpu.VMEM((C,), jnp.int32),
        pltpu.VMEM((C,), jnp.int32),
        pltpu.VMEM((C,), jnp.int32),
        pltpu.VMEM((C,), jnp.int32),
        pltpu.VMEM((C,), jnp.int32),
        pltpu.VMEM((C,), jnp.int32),
        pltpu.VMEM((TAILE,), jnp.int32),
        pltpu.VMEM((TAILE,), jnp.int32),
        pltpu.VMEM((C, DOUT), jnp.float32),
        pltpu.VMEM((C, DOUT), jnp.float32),
        pltpu.VMEM((C, DOUT), jnp.float32),
        pltpu.VMEM((C, DOUT), jnp.float32),
        pltpu.VMEM((C, DOUT), jnp.float32),
        pltpu.VMEM((C, DOUT), jnp.float32),
        pltpu.VMEM_SHARED((N, DOUT), jnp.float32),
        pltpu.SemaphoreType.DMA,
        pltpu.SemaphoreType.DMA,
        pltpu.SemaphoreType.DMA,
        pltpu.SemaphoreType.DMA,
        pltpu.SemaphoreType.DMA,
        pltpu.SemaphoreType.DMA,
        pltpu.SemaphoreType.DMA,
        pltpu.SemaphoreType.DMA,
        pltpu.SemaphoreType.DMA,
        pltpu.SemaphoreType.DMA,
        pltpu.SemaphoreType.DMA,
        pltpu.SemaphoreType.DMA,
    ],
)
def _edge_kernel(src_hbm, dst_hbm, psrc_hbm, pdst_hbm, patt_hbm,
                 eout_hbm, agg_hbm,
                 idx_s0, idx_d0, idx_s1, idx_d1, sidx0, sidx1,
                 idx_st, idx_dt,
                 a0, b0, c0, a1, b1, c1, agg_sh,
                 ga_sem0, gb_sem0, pc_sem0, ga_sem1, gb_sem1, pc_sem1,
                 eo_sem0, sc_sem0, eo_sem1, sc_sem1, ix_sem0, ix_sem1):
    cid = lax.axis_index("c")
    sid = lax.axis_index("s")
    wid = sid * NC + cid
    base_w = wid * EPW

    sets = (
        dict(idx_s=idx_s0, idx_d=idx_d0, a=a0, b=b0, c=c0, sidx=sidx0,
             ga=ga_sem0, gb=gb_sem0, pc=pc_sem0, eo=eo_sem0, sc=sc_sem0,
             ix=ix_sem0),
        dict(idx_s=idx_s1, idx_d=idx_d1, a=a1, b=b1, c=c1, sidx=sidx1,
             ga=ga_sem1, gb=gb_sem1, pc=pc_sem1, eo=eo_sem1, sc=sc_sem1,
             ix=ix_sem1),
    )

    def idx_descs(base, s):
        t = sets[s]
        return (pltpu.make_async_copy(src_hbm.at[pl.ds(base, C)],
                                      t["idx_s"], t["ix"]),
                pltpu.make_async_copy(dst_hbm.at[pl.ds(base, C)],
                                      t["idx_d"], t["ix"]))

    def in_descs(base, s):
        t = sets[s]
        return (pltpu.make_async_copy(psrc_hbm.at[t["idx_s"]],
                                      t["a"], t["ga"]),
                pltpu.make_async_copy(pdst_hbm.at[t["idx_d"]],
                                      t["b"], t["gb"]),
                pltpu.make_async_copy(patt_hbm.at[pl.ds(base, C)],
                                      t["c"], t["pc"]))

    def out_descs(base, s):
        t = sets[s]
        return (pltpu.make_async_copy(t["a"],
                                      eout_hbm.at[pl.ds(base, C)],
                                      t["eo"]),
                pltpu.make_async_copy(t["a"],
                                      agg_sh.at[t["sidx"]], t["sc"]))

    def snap_idx(s):
        t = sets[s]
        for g in range(C // L):
            sl = pl.ds(g * L, L)
            t["sidx"][sl] = t["idx_d"][sl]

    def fire_out(base, s):
        d = out_descs(base, s)
        d[0].start()
        d[1].start(add=True)

    def _compute(a, b, c, nrows):
        def _row(i, rcarry):
            for g in range(DOUT // L):
                sl = pl.ds(g * L, L)
                a[i, sl] = jnp.maximum(a[i, sl] + b[i, sl] + c[i, sl], 0.0)
            return rcarry

        lax.fori_loop(0, nrows, _row, 0)

    def _zrow(i, carry):
        for g in range(DOUT // L):
            a0[i, pl.ds(g * L, L)] = jnp.zeros((L,), jnp.float32)
        return carry

    lax.fori_loop(0, C, _zrow, 0)
    nblk_me = jnp.where(sid < NBLK - 9 * NS, 10, 9)

    def _zblk(k, carry):
        pltpu.sync_copy(a0, agg_sh.at[pl.ds((sid + NS * k) * C, C)])
        return carry

    lax.fori_loop(0, nblk_me, _zblk, 0)

    @pl.when(sid == NS - 1)
    def _zero_tail():
        pltpu.sync_copy(a0.at[pl.ds(0, AGG_TAILR)],
                        agg_sh.at[pl.ds(AGG_TAIL0, AGG_TAILR)])

    plsc.subcore_barrier()

    pltpu.sync_copy(src_hbm.at[pl.ds(base_w, C)], idx_s0)
    pltpu.sync_copy(dst_hbm.at[pl.ds(base_w, C)], idx_d0)
    for d in in_descs(base_w, 0):
        d.start()
    for d in idx_descs(base_w + C, 1):
        d.start()

    def _pair(p, carry):
        base_a = base_w + 2 * p * C

        @pl.when(p > 0)
        def _():
            for d in out_descs(base_a - C, 1):
                d.wait()

        for d in idx_descs(base_a + C, 1):
            d.wait()
        for d in in_descs(base_a + C, 1):
            d.start()
        for d in in_descs(base_a, 0):
            d.wait()
        snap_idx(0)

        @pl.when(p < PAIRS - 1)
        def _():
            for d in idx_descs(base_a + 2 * C, 0):
                d.start()

        _compute(a0, b0, c0, C)
        fire_out(base_a, 0)

        base_b = base_a + C
        for d in out_descs(base_a, 0):
            d.wait()

        @pl.when(p < PAIRS - 1)
        def _():
            for d in idx_descs(base_b + C, 0):
                d.wait()
            for d in in_descs(base_b + C, 0):
                d.start()

        for d in in_descs(base_b, 1):
            d.wait()
        snap_idx(1)

        @pl.when(p < PAIRS - 1)
        def _():
            for d in idx_descs(base_b + 2 * C, 1):
                d.start()

        _compute(a1, b1, c1, C)
        fire_out(base_b, 1)
        return carry

    lax.fori_loop(0, PAIRS, _pair, 0)
    for d in out_descs(base_w + (NCH - 1) * C, 1):
        d.wait()

    tb = base_w + TBASE
    pltpu.sync_copy(src_hbm.at[pl.ds(tb, TAILE)], idx_st)
    pltpu.sync_copy(dst_hbm.at[pl.ds(tb, TAILE)], idx_dt)
    cp1 = pltpu.async_copy(psrc_hbm.at[idx_st], a0.at[pl.ds(0, TAILE)],
                           ga_sem0)
    cp2 = pltpu.async_copy(pdst_hbm.at[idx_dt], b0.at[pl.ds(0, TAILE)],
                           gb_sem0)
    cp3 = pltpu.async_copy(patt_hbm.at[pl.ds(tb, TAILE)],
                           c0.at[pl.ds(0, TAILE)], pc_sem0)
    cp1.wait()
    cp2.wait()
    cp3.wait()
    _compute(a0, b0, c0, TAILE)
    tr = a0.at[pl.ds(0, TAILE)]
    pltpu.sync_copy(tr, eout_hbm.at[pl.ds(tb, TAILE)])
    pltpu.sync_copy(tr, agg_sh.at[idx_dt], add=True)

    plsc.subcore_barrier()

    def _dblk(k, carry):
        off = (sid + NS * k) * C
        pltpu.sync_copy(agg_sh.at[pl.ds(off, C)], a0)
        pltpu.sync_copy(a0, agg_hbm.at[cid, pl.ds(off, C)])
        return carry

    lax.fori_loop(0, nblk_me, _dblk, 0)

    @pl.when(sid == NS - 1)
    def _drain_tail():
        pltpu.sync_copy(agg_sh.at[pl.ds(AGG_TAIL0, AGG_TAILR)],
                        a0.at[pl.ds(0, AGG_TAILR)])
        pltpu.sync_copy(a0.at[pl.ds(0, AGG_TAILR)],
                        agg_hbm.at[cid, pl.ds(AGG_TAIL0, AGG_TAILR)])


@jax.jit
def kernel(x, edge_index, edge_attr, W_e, b_e, W_n, b_n):
    src = edge_index[0]
    dst = edge_index[1]
    psrc, pdst = _proj(x, W_e[:D], W_e[D:2 * D])
    patt = _patt(edge_attr.T, W_e[2 * D:], b_e.reshape(1, DOUT))
    edge_out, aggs = _edge_kernel(src, dst, psrc, pdst, patt)
    x_out = _node(x, aggs, W_n[:D], W_n[D:], b_n.reshape(1, D))
    return (x_out, edge_out)


# speedup vs baseline: 1.3159x; 1.0053x over previous
import functools

import jax
import jax.numpy as jnp
from jax import lax
from jax.experimental import pallas as pl
from jax.experimental.pallas import tpu as pltpu
from jax.experimental.pallas import tpu_sc as plsc

N = 10000
E = 320000
D = 128
DE = 16
DOUT = 128

NC = 2
NS = 16
L = 16
NW = NC * NS
EPW = E // NW
C = 64
NCH = 156
PAIRS = NCH // 2
TAILE = EPW - NCH * C
TBASE = NCH * C
NBLK = N // C
AGG_TAIL0 = NBLK * C
AGG_TAILR = N - AGG_TAIL0


def _proj_body(x_ref, w1_ref, w2_ref, o1_ref, o2_ref):
    xb = x_ref[...]
    o1_ref[...] = jnp.dot(xb, w1_ref[...], preferred_element_type=jnp.float32)
    o2_ref[...] = jnp.dot(xb, w2_ref[...], preferred_element_type=jnp.float32)


def _proj(x, w1, w2):
    bn = 2000
    grid = N // bn
    return pl.pallas_call(
        _proj_body,
        grid=(grid,),
        in_specs=[
            pl.BlockSpec((bn, D), lambda i: (i, 0)),
            pl.BlockSpec((D, D), lambda i: (0, 0)),
            pl.BlockSpec((D, D), lambda i: (0, 0)),
        ],
        out_specs=[
            pl.BlockSpec((bn, D), lambda i: (i, 0)),
            pl.BlockSpec((bn, D), lambda i: (i, 0)),
        ],
        out_shape=[
            jax.ShapeDtypeStruct((N, D), jnp.float32),
            jax.ShapeDtypeStruct((N, D), jnp.float32),
        ],
    )(x, w1, w2)


def _patt_body(at_ref, w_ref, b_ref, o_ref):
    o_ref[...] = lax.dot_general(
        at_ref[...], w_ref[...],
        dimension_numbers=(((0,), (0,)), ((), ())),
        preferred_element_type=jnp.float32) + b_ref[...]


def _patt(edge_attr_t, w3, b_e):
    be = 6400
    grid = E // be
    return pl.pallas_call(
        _patt_body,
        grid=(grid,),
        in_specs=[
            pl.BlockSpec((DE, be), lambda i: (0, i)),
            pl.BlockSpec((DE, DOUT), lambda i: (0, 0)),
            pl.BlockSpec((1, DOUT), lambda i: (0, 0)),
        ],
        out_specs=pl.BlockSpec((be, DOUT), lambda i: (i, 0)),
        out_shape=jax.ShapeDtypeStruct((E, DOUT), jnp.float32),
    )(edge_attr_t, w3, b_e)


def _node_body(x_ref, a_ref, w1_ref, w2_ref, b_ref, o_ref):
    acc = jnp.dot(x_ref[...], w1_ref[...], preferred_element_type=jnp.float32)
    acc += jnp.dot(a_ref[0] + a_ref[1], w2_ref[...],
                   preferred_element_type=jnp.float32)
    o_ref[...] = jnp.maximum(acc + b_ref[...], 0.0)


def _node(x, aggs, wn1, wn2, b_n):
    bn = 1000
    grid = N // bn
    return pl.pallas_call(
        _node_body,
        grid=(grid,),
        in_specs=[
            pl.BlockSpec((bn, D), lambda i: (i, 0)),
            pl.BlockSpec((NC, bn, DOUT), lambda i: (0, i, 0)),
            pl.BlockSpec((D, D), lambda i: (0, 0)),
            pl.BlockSpec((DOUT, D), lambda i: (0, 0)),
            pl.BlockSpec((1, D), lambda i: (0, 0)),
        ],
        out_specs=pl.BlockSpec((bn, D), lambda i: (i, 0)),
        out_shape=jax.ShapeDtypeStruct((N, D), jnp.float32),
    )(x, aggs, wn1, wn2, b_n)


_sc_mesh = plsc.VectorSubcoreMesh(core_axis_name="c", subcore_axis_name="s")


@functools.partial(
    pl.kernel,
    out_type=(
        jax.ShapeDtypeStruct((E, DOUT), jnp.float32),
        jax.ShapeDtypeStruct((NC, N, DOUT), jnp.float32),
    ),
    mesh=_sc_mesh,
    scratch_types=[
        pltpu.VMEM((C,), jnp.int32),
        pltpu.VMEM((C,), jnp.int32),
        pltpu.VMEM((C,), jnp.int32),
        pltpu.VMEM((C,), jnp.int32),
        pltpu.VMEM((C,), jnp.int32),
        pltpu.VMEM((C,), jnp.int32),
        pltpu.VMEM((TAILE,), jnp.int32),
        pltpu.VMEM((TAILE,), jnp.int32),
        pltpu.VMEM((C, DOUT), jnp.float32),
        pltpu.VMEM((C, DOUT), jnp.float32),
        pltpu.VMEM((C, DOUT), jnp.float32),
        pltpu.VMEM((C, DOUT), jnp.float32),
        pltpu.VMEM((C, DOUT), jnp.float32),
        pltpu.VMEM((C, DOUT), jnp.float32),
        pltpu.VMEM_SHARED((N, DOUT), jnp.float32),
        pltpu.SemaphoreType.DMA,
        pltpu.SemaphoreType.DMA,
        pltpu.SemaphoreType.DMA,
        pltpu.SemaphoreType.DMA,
        pltpu.SemaphoreType.DMA,
        pltpu.SemaphoreType.DMA,
        pltpu.SemaphoreType.DMA,
        pltpu.SemaphoreType.DMA,
        pltpu.SemaphoreType.DMA,
        pltpu.SemaphoreType.DMA,
        pltpu.SemaphoreType.DMA,
        pltpu.SemaphoreType.DMA,
    ],
)
def _edge_kernel(src_hbm, dst_hbm, psrc_hbm, pdst_hbm, patt_hbm,
                 eout_hbm, agg_hbm,
                 idx_s0, idx_d0, idx_s1, idx_d1, sidx0, sidx1,
                 idx_st, idx_dt,
                 a0, b0, c0, a1, b1, c1, agg_sh,
                 ga_sem0, gb_sem0, pc_sem0, ga_sem1, gb_sem1, pc_sem1,
                 eo_sem0, sc_sem0, eo_sem1, sc_sem1, ix_sem0, ix_sem1):
    cid = lax.axis_index("c")
    sid = lax.axis_index("s")
    wid = sid * NC + cid
    base_w = wid * EPW

    sets = (
        dict(idx_s=idx_s0, idx_d=idx_d0, a=a0, b=b0, c=c0, sidx=sidx0,
             ga=ga_sem0, gb=gb_sem0, pc=pc_sem0, eo=eo_sem0, sc=sc_sem0,
             ix=ix_sem0),
        dict(idx_s=idx_s1, idx_d=idx_d1, a=a1, b=b1, c=c1, sidx=sidx1,
             ga=ga_sem1, gb=gb_sem1, pc=pc_sem1, eo=eo_sem1, sc=sc_sem1,
             ix=ix_sem1),
    )

    def idx_descs(base, s):
        t = sets[s]
        return (pltpu.make_async_copy(src_hbm.at[pl.ds(base, C)],
                                      t["idx_s"], t["ix"]),
                pltpu.make_async_copy(dst_hbm.at[pl.ds(base, C)],
                                      t["idx_d"], t["ix"]))

    def in_descs(base, s):
        t = sets[s]
        return (pltpu.make_async_copy(psrc_hbm.at[t["idx_s"]],
                                      t["a"], t["ga"]),
                pltpu.make_async_copy(pdst_hbm.at[t["idx_d"]],
                                      t["b"], t["gb"]),
                pltpu.make_async_copy(patt_hbm.at[pl.ds(base, C)],
                                      t["c"], t["pc"]))

    def out_descs(base, s):
        t = sets[s]
        return (pltpu.make_async_copy(t["a"],
                                      eout_hbm.at[pl.ds(base, C)],
                                      t["eo"]),
                pltpu.make_async_copy(t["a"],
                                      agg_sh.at[t["sidx"]], t["sc"]))

    def snap_idx(s):
        t = sets[s]
        for g in range(C // L):
            sl = pl.ds(g * L, L)
            t["sidx"][sl] = t["idx_d"][sl]

    def fire_out(base, s):
        d = out_descs(base, s)
        d[0].start()
        d[1].start(add=True)

    def _compute(a, b, c, nrows):
        def _row(i, rcarry):
            for u in range(2):
                for g in range(DOUT // L):
                    sl = pl.ds(g * L, L)
                    r = 2 * i + u
                    a[r, sl] = jnp.maximum(
                        a[r, sl] + b[r, sl] + c[r, sl], 0.0)
            return rcarry

        lax.fori_loop(0, nrows // 2, _row, 0)

    for d in idx_descs(base_w, 0):
        d.start()

    def _zrow(i, carry):
        for g in range(DOUT // L):
            c1[i, pl.ds(g * L, L)] = jnp.zeros((L,), jnp.float32)
        return carry

    lax.fori_loop(0, C, _zrow, 0)
    for d in idx_descs(base_w, 0):
        d.wait()
    for d in in_descs(base_w, 0):
        d.start()
    for d in idx_descs(base_w + C, 1):
        d.start()

    nblk_me = jnp.where(sid < NBLK - 9 * NS, 10, 9)

    def _zblk(k, carry):
        pltpu.sync_copy(c1, agg_sh.at[pl.ds((sid + NS * k) * C, C)])
        return carry

    lax.fori_loop(0, nblk_me, _zblk, 0)

    @pl.when(sid == NS - 1)
    def _zero_tail():
        pltpu.sync_copy(c1.at[pl.ds(0, AGG_TAILR)],
                        agg_sh.at[pl.ds(AGG_TAIL0, AGG_TAILR)])

    plsc.subcore_barrier()

    def _pair(p, carry):
        base_a = base_w + 2 * p * C

        @pl.when(p > 0)
        def _():
            for d in out_descs(base_a - C, 1):
                d.wait()

        for d in idx_descs(base_a + C, 1):
            d.wait()
        for d in in_descs(base_a + C, 1):
            d.start()
        for d in in_descs(base_a, 0):
            d.wait()
        snap_idx(0)

        @pl.when(p < PAIRS - 1)
        def _():
            for d in idx_descs(base_a + 2 * C, 0):
                d.start()

        _compute(a0, b0, c0, C)
        fire_out(base_a, 0)

        base_b = base_a + C
        for d in out_descs(base_a, 0):
            d.wait()

        @pl.when(p < PAIRS - 1)
        def _():
            for d in idx_descs(base_b + C, 0):
                d.wait()
            for d in in_descs(base_b + C, 0):
                d.start()

        for d in in_descs(base_b, 1):
            d.wait()
        snap_idx(1)

        @pl.when(p < PAIRS - 1)
        def _():
            for d in idx_descs(base_b + 2 * C, 1):
                d.start()

        _compute(a1, b1, c1, C)
        fire_out(base_b, 1)
        return carry

    lax.fori_loop(0, PAIRS, _pair, 0)
    for d in out_descs(base_w + (NCH - 1) * C, 1):
        d.wait()

    tb = base_w + TBASE
    pltpu.sync_copy(src_hbm.at[pl.ds(tb, TAILE)], idx_st)
    pltpu.sync_copy(dst_hbm.at[pl.ds(tb, TAILE)], idx_dt)
    cp1 = pltpu.async_copy(psrc_hbm.at[idx_st], a0.at[pl.ds(0, TAILE)],
                           ga_sem0)
    cp2 = pltpu.async_copy(pdst_hbm.at[idx_dt], b0.at[pl.ds(0, TAILE)],
                           gb_sem0)
    cp3 = pltpu.async_copy(patt_hbm.at[pl.ds(tb, TAILE)],
                           c0.at[pl.ds(0, TAILE)], pc_sem0)
    cp1.wait()
    cp2.wait()
    cp3.wait()
    _compute(a0, b0, c0, TAILE)
    tr = a0.at[pl.ds(0, TAILE)]
    pltpu.sync_copy(tr, eout_hbm.at[pl.ds(tb, TAILE)])
    pltpu.sync_copy(tr, agg_sh.at[idx_dt], add=True)

    plsc.subcore_barrier()

    def _dblk(k, carry):
        off = (sid + NS * k) * C
        pltpu.sync_copy(agg_sh.at[pl.ds(off, C)], a0)
        pltpu.sync_copy(a0, agg_hbm.at[cid, pl.ds(off, C)])
        return carry

    lax.fori_loop(0, nblk_me, _dblk, 0)

    @pl.when(sid == NS - 1)
    def _drain_tail():
        pltpu.sync_copy(agg_sh.at[pl.ds(AGG_TAIL0, AGG_TAILR)],
                        a0.at[pl.ds(0, AGG_TAILR)])
        pltpu.sync_copy(a0.at[pl.ds(0, AGG_TAILR)],
                        agg_hbm.at[cid, pl.ds(AGG_TAIL0, AGG_TAILR)])


@jax.jit
def kernel(x, edge_index, edge_attr, W_e, b_e, W_n, b_n):
    src = edge_index[0]
    dst = edge_index[1]
    psrc, pdst = _proj(x, W_e[:D], W_e[D:2 * D])
    patt = _patt(edge_attr.T, W_e[2 * D:], b_e.reshape(1, DOUT))
    edge_out, aggs = _edge_kernel(src, dst, psrc, pdst, patt)
    x_out = _node(x, aggs, W_n[:D], W_n[D:], b_n.reshape(1, D))
    return (x_out, edge_out)


# speedup vs baseline: 1.3734x; 1.0437x over previous
import functools

import jax
import jax.numpy as jnp
from jax import lax
from jax.experimental import pallas as pl
from jax.experimental.pallas import tpu as pltpu
from jax.experimental.pallas import tpu_sc as plsc

N = 10000
E = 320000
D = 128
DE = 16
DOUT = 128

NC = 2
NS = 16
L = 16
NW = NC * NS
EPW = E // NW
C = 64
NCH = 156
PAIRS = NCH // 2
TAILE = EPW - NCH * C
TBASE = NCH * C
NBLK = N // C
AGG_TAIL0 = NBLK * C
AGG_TAILR = N - AGG_TAIL0


def _proj_body(x_ref, w1_ref, w2_ref, o1_ref, o2_ref):
    xb = x_ref[...]
    o1_ref[...] = jnp.dot(xb, w1_ref[...], preferred_element_type=jnp.float32)
    o2_ref[...] = jnp.dot(xb, w2_ref[...], preferred_element_type=jnp.float32)


def _proj(x, w1, w2):
    bn = 2000
    grid = N // bn
    return pl.pallas_call(
        _proj_body,
        grid=(grid,),
        in_specs=[
            pl.BlockSpec((bn, D), lambda i: (i, 0)),
            pl.BlockSpec((D, D), lambda i: (0, 0)),
            pl.BlockSpec((D, D), lambda i: (0, 0)),
        ],
        out_specs=[
            pl.BlockSpec((bn, D), lambda i: (i, 0)),
            pl.BlockSpec((bn, D), lambda i: (i, 0)),
        ],
        out_shape=[
            jax.ShapeDtypeStruct((N, D), jnp.float32),
            jax.ShapeDtypeStruct((N, D), jnp.float32),
        ],
    )(x, w1, w2)


def _patt_body(at_ref, w_ref, b_ref, o_ref):
    o_ref[...] = lax.dot_general(
        at_ref[...], w_ref[...],
        dimension_numbers=(((0,), (0,)), ((), ())),
        preferred_element_type=jnp.float32) + b_ref[...]


def _patt(edge_attr_t, w3, b_e):
    be = 12800
    grid = E // be
    return pl.pallas_call(
        _patt_body,
        grid=(grid,),
        in_specs=[
            pl.BlockSpec((DE, be), lambda i: (0, i)),
            pl.BlockSpec((DE, DOUT), lambda i: (0, 0)),
            pl.BlockSpec((1, DOUT), lambda i: (0, 0)),
        ],
        out_specs=pl.BlockSpec((be, DOUT), lambda i: (i, 0)),
        out_shape=jax.ShapeDtypeStruct((E, DOUT), jnp.float32),
    )(edge_attr_t, w3, b_e)


def _node_body(x_ref, a_ref, w1_ref, w2_ref, b_ref, o_ref):
    acc = jnp.dot(x_ref[...], w1_ref[...], preferred_element_type=jnp.float32)
    acc += jnp.dot(a_ref[0] + a_ref[1], w2_ref[...],
                   preferred_element_type=jnp.float32)
    o_ref[...] = jnp.maximum(acc + b_ref[...], 0.0)


def _node(x, aggs, wn1, wn2, b_n):
    bn = 2000
    grid = N // bn
    return pl.pallas_call(
        _node_body,
        grid=(grid,),
        in_specs=[
            pl.BlockSpec((bn, D), lambda i: (i, 0)),
            pl.BlockSpec((NC, bn, DOUT), lambda i: (0, i, 0)),
            pl.BlockSpec((D, D), lambda i: (0, 0)),
            pl.BlockSpec((DOUT, D), lambda i: (0, 0)),
            pl.BlockSpec((1, D), lambda i: (0, 0)),
        ],
        out_specs=pl.BlockSpec((bn, D), lambda i: (i, 0)),
        out_shape=jax.ShapeDtypeStruct((N, D), jnp.float32),
    )(x, aggs, wn1, wn2, b_n)


_sc_mesh = plsc.VectorSubcoreMesh(core_axis_name="c", subcore_axis_name="s")


@functools.partial(
    pl.kernel,
    out_type=(
        jax.ShapeDtypeStruct((E, DOUT), jnp.float32),
        jax.ShapeDtypeStruct((NC, N, DOUT), jnp.float32),
    ),
    mesh=_sc_mesh,
    scratch_types=[
        pltpu.VMEM((C,), jnp.int32),
        pltpu.VMEM((C,), jnp.int32),
        pltpu.VMEM((C,), jnp.int32),
        pltpu.VMEM((C,), jnp.int32),
        pltpu.VMEM((C,), jnp.int32),
        pltpu.VMEM((C,), jnp.int32),
        pltpu.VMEM((TAILE,), jnp.int32),
        pltpu.VMEM((TAILE,), jnp.int32),
        pltpu.VMEM((C, DOUT), jnp.float32),
        pltpu.VMEM((C, DOUT), jnp.float32),
        pltpu.VMEM((C, DOUT), jnp.float32),
        pltpu.VMEM((C, DOUT), jnp.float32),
        pltpu.VMEM((C, DOUT), jnp.float32),
        pltpu.VMEM((C, DOUT), jnp.float32),
        pltpu.VMEM_SHARED((N, DOUT), jnp.float32),
        pltpu.SemaphoreType.DMA,
        pltpu.SemaphoreType.DMA,
        pltpu.SemaphoreType.DMA,
        pltpu.SemaphoreType.DMA,
        pltpu.SemaphoreType.DMA,
        pltpu.SemaphoreType.DMA,
        pltpu.SemaphoreType.DMA,
        pltpu.SemaphoreType.DMA,
        pltpu.SemaphoreType.DMA,
        pltpu.SemaphoreType.DMA,
        pltpu.SemaphoreType.DMA,
        pltpu.SemaphoreType.DMA,
    ],
)
def _edge_kernel(src_hbm, dst_hbm, psrc_hbm, pdst_hbm, patt_hbm,
                 eout_hbm, agg_hbm,
                 idx_s0, idx_d0, idx_s1, idx_d1, sidx0, sidx1,
                 idx_st, idx_dt,
                 a0, b0, c0, a1, b1, c1, agg_sh,
                 ga_sem0, gb_sem0, pc_sem0, ga_sem1, gb_sem1, pc_sem1,
                 eo_sem0, sc_sem0, eo_sem1, sc_sem1, ix_sem0, ix_sem1):
    cid = lax.axis_index("c")
    sid = lax.axis_index("s")
    wid = sid * NC + cid
    base_w = wid * EPW

    sets = (
        dict(idx_s=idx_s0, idx_d=idx_d0, a=a0, b=b0, c=c0, sidx=sidx0,
             ga=ga_sem0, gb=gb_sem0, pc=pc_sem0, eo=eo_sem0, sc=sc_sem0,
             ix=ix_sem0),
        dict(idx_s=idx_s1, idx_d=idx_d1, a=a1, b=b1, c=c1, sidx=sidx1,
             ga=ga_sem1, gb=gb_sem1, pc=pc_sem1, eo=eo_sem1, sc=sc_sem1,
             ix=ix_sem1),
    )

    def idx_descs(base, s):
        t = sets[s]
        return (pltpu.make_async_copy(src_hbm.at[pl.ds(base, C)],
                                      t["idx_s"], t["ix"]),
                pltpu.make_async_copy(dst_hbm.at[pl.ds(base, C)],
                                      t["idx_d"], t["ix"]))

    def in_descs(base, s):
        t = sets[s]
        return (pltpu.make_async_copy(psrc_hbm.at[t["idx_s"]],
                                      t["a"], t["ga"]),
                pltpu.make_async_copy(pdst_hbm.at[t["idx_d"]],
                                      t["b"], t["gb"]),
                pltpu.make_async_copy(patt_hbm.at[pl.ds(base, C)],
                                      t["c"], t["pc"]))

    def out_descs(base, s):
        t = sets[s]
        return (pltpu.make_async_copy(t["a"],
                                      eout_hbm.at[pl.ds(base, C)],
                                      t["eo"]),
                pltpu.make_async_copy(t["a"],
                                      agg_sh.at[t["sidx"]], t["sc"]))

    def snap_idx(s):
        t = sets[s]
        for g in range(C // L):
            sl = pl.ds(g * L, L)
            t["sidx"][sl] = t["idx_d"][sl]

    def fire_out(base, s):
        d = out_descs(base, s)
        d[0].start()
        d[1].start(add=True)

    def _compute(a, b, c, nrows):
        def _row(i, rcarry):
            for u in range(2):
                for g in range(DOUT // L):
                    sl = pl.ds(g * L, L)
                    r = 2 * i + u
                    a[r, sl] = jnp.maximum(
                        a[r, sl] + b[r, sl] + c[r, sl], 0.0)
            return rcarry

        lax.fori_loop(0, nrows // 2, _row, 0)

    for d in idx_descs(base_w, 0):
        d.start()

    def _zrow(i, carry):
        for g in range(DOUT // L):
            c1[i, pl.ds(g * L, L)] = jnp.zeros((L,), jnp.float32)
        return carry

    lax.fori_loop(0, C, _zrow, 0)
    for d in idx_descs(base_w, 0):
        d.wait()
    for d in in_descs(base_w, 0):
        d.start()
    for d in idx_descs(base_w + C, 1):
        d.start()

    nblk_me = jnp.where(sid < NBLK - 9 * NS, 10, 9)

    def _zblk(k, carry):
        pltpu.sync_copy(c1, agg_sh.at[pl.ds((sid + NS * k) * C, C)])
        return carry

    lax.fori_loop(0, nblk_me, _zblk, 0)

    @pl.when(sid == NS - 1)
    def _zero_tail():
        pltpu.sync_copy(c1.at[pl.ds(0, AGG_TAILR)],
                        agg_sh.at[pl.ds(AGG_TAIL0, AGG_TAILR)])

    plsc.subcore_barrier()

    def _pair(p, carry):
        base_a = base_w + 2 * p * C

        @pl.when(p > 0)
        def _():
            for d in out_descs(base_a - C, 1):
                d.wait()

        for d in idx_descs(base_a + C, 1):
            d.wait()
        for d in in_descs(base_a + C, 1):
            d.start()
        for d in in_descs(base_a, 0):
            d.wait()
        snap_idx(0)

        @pl.when(p < PAIRS - 1)
        def _():
            for d in idx_descs(base_a + 2 * C, 0):
                d.start()

        _compute(a0, b0, c0, C)
        fire_out(base_a, 0)

        base_b = base_a + C
        for d in out_descs(base_a, 0):
            d.wait()

        @pl.when(p < PAIRS - 1)
        def _():
            for d in idx_descs(base_b + C, 0):
                d.wait()
            for d in in_descs(base_b + C, 0):
                d.start()

        for d in in_descs(base_b, 1):
            d.wait()
        snap_idx(1)

        @pl.when(p < PAIRS - 1)
        def _():
            for d in idx_descs(base_b + 2 * C, 1):
                d.start()

        _compute(a1, b1, c1, C)
        fire_out(base_b, 1)
        return carry

    lax.fori_loop(0, PAIRS, _pair, 0)
    for d in out_descs(base_w + (NCH - 1) * C, 1):
        d.wait()

    tb = base_w + TBASE
    pltpu.sync_copy(src_hbm.at[pl.ds(tb, TAILE)], idx_st)
    pltpu.sync_copy(dst_hbm.at[pl.ds(tb, TAILE)], idx_dt)
    cp1 = pltpu.async_copy(psrc_hbm.at[idx_st], a0.at[pl.ds(0, TAILE)],
                           ga_sem0)
    cp2 = pltpu.async_copy(pdst_hbm.at[idx_dt], b0.at[pl.ds(0, TAILE)],
                           gb_sem0)
    cp3 = pltpu.async_copy(patt_hbm.at[pl.ds(tb, TAILE)],
                           c0.at[pl.ds(0, TAILE)], pc_sem0)
    cp1.wait()
    cp2.wait()
    cp3.wait()
    _compute(a0, b0, c0, TAILE)
    tr = a0.at[pl.ds(0, TAILE)]
    pltpu.sync_copy(tr, eout_hbm.at[pl.ds(tb, TAILE)])
    pltpu.sync_copy(tr, agg_sh.at[idx_dt], add=True)

    plsc.subcore_barrier()

    def _dblk(k, carry):
        off = (sid + NS * k) * C
        pltpu.sync_copy(agg_sh.at[pl.ds(off, C)], a0)
        pltpu.sync_copy(a0, agg_hbm.at[cid, pl.ds(off, C)])
        return carry

    lax.fori_loop(0, nblk_me, _dblk, 0)

    @pl.when(sid == NS - 1)
    def _drain_tail():
        pltpu.sync_copy(agg_sh.at[pl.ds(AGG_TAIL0, AGG_TAILR)],
                        a0.at[pl.ds(0, AGG_TAILR)])
        pltpu.sync_copy(a0.at[pl.ds(0, AGG_TAILR)],
                        agg_hbm.at[cid, pl.ds(AGG_TAIL0, AGG_TAILR)])


@jax.jit
def kernel(x, edge_index, edge_attr, W_e, b_e, W_n, b_n):
    src = edge_index[0]
    dst = edge_index[1]
    psrc, pdst = _proj(x, W_e[:D], W_e[D:2 * D])
    patt = _patt(edge_attr.T, W_e[2 * D:], b_e.reshape(1, DOUT))
    edge_out, aggs = _edge_kernel(src, dst, psrc, pdst, patt)
    x_out = _node(x, aggs, W_n[:D], W_n[D:], b_n.reshape(1, D))
    return (x_out, edge_out)


# speedup vs baseline: 1.3819x; 1.0062x over previous
import functools

import jax
import jax.numpy as jnp
from jax import lax
from jax.experimental import pallas as pl
from jax.experimental.pallas import tpu as pltpu
from jax.experimental.pallas import tpu_sc as plsc

N = 10000
E = 320000
D = 128
DE = 16
DOUT = 128

NC = 2
NS = 16
L = 16
NW = NC * NS
EPW = E // NW
C = 64
NCH = 156
PAIRS = NCH // 2
TAILE = EPW - NCH * C
TBASE = NCH * C
NBLK = N // C
AGG_TAIL0 = NBLK * C
AGG_TAILR = N - AGG_TAIL0


def _proj_body(x_ref, w1_ref, w2_ref, o1_ref, o2_ref):
    xb = x_ref[...]
    o1_ref[...] = jnp.dot(xb, w1_ref[...], preferred_element_type=jnp.float32)
    o2_ref[...] = jnp.dot(xb, w2_ref[...], preferred_element_type=jnp.float32)


def _proj(x, w1, w2):
    bn = 2000
    grid = N // bn
    return pl.pallas_call(
        _proj_body,
        grid=(grid,),
        in_specs=[
            pl.BlockSpec((bn, D), lambda i: (i, 0)),
            pl.BlockSpec((D, D), lambda i: (0, 0)),
            pl.BlockSpec((D, D), lambda i: (0, 0)),
        ],
        out_specs=[
            pl.BlockSpec((bn, D), lambda i: (i, 0)),
            pl.BlockSpec((bn, D), lambda i: (i, 0)),
        ],
        out_shape=[
            jax.ShapeDtypeStruct((N, D), jnp.float32),
            jax.ShapeDtypeStruct((N, D), jnp.float32),
        ],
    )(x, w1, w2)


def _patt_body(at_ref, w_ref, b_ref, o_ref):
    o_ref[...] = lax.dot_general(
        at_ref[...], w_ref[...],
        dimension_numbers=(((0,), (0,)), ((), ())),
        preferred_element_type=jnp.float32) + b_ref[...]


def _patt(edge_attr_t, w3, b_e):
    be = 16000
    grid = E // be
    return pl.pallas_call(
        _patt_body,
        grid=(grid,),
        in_specs=[
            pl.BlockSpec((DE, be), lambda i: (0, i)),
            pl.BlockSpec((DE, DOUT), lambda i: (0, 0)),
            pl.BlockSpec((1, DOUT), lambda i: (0, 0)),
        ],
        out_specs=pl.BlockSpec((be, DOUT), lambda i: (i, 0)),
        out_shape=jax.ShapeDtypeStruct((E, DOUT), jnp.float32),
    )(edge_attr_t, w3, b_e)


def _node_body(x_ref, a_ref, w1_ref, w2_ref, b_ref, o_ref):
    acc = jnp.dot(x_ref[...], w1_ref[...], preferred_element_type=jnp.float32)
    acc += jnp.dot(a_ref[0] + a_ref[1], w2_ref[...],
                   preferred_element_type=jnp.float32)
    o_ref[...] = jnp.maximum(acc + b_ref[...], 0.0)


def _node(x, aggs, wn1, wn2, b_n):
    bn = 2000
    grid = N // bn
    return pl.pallas_call(
        _node_body,
        grid=(grid,),
        in_specs=[
            pl.BlockSpec((bn, D), lambda i: (i, 0)),
            pl.BlockSpec((NC, bn, DOUT), lambda i: (0, i, 0)),
            pl.BlockSpec((D, D), lambda i: (0, 0)),
            pl.BlockSpec((DOUT, D), lambda i: (0, 0)),
            pl.BlockSpec((1, D), lambda i: (0, 0)),
        ],
        out_specs=pl.BlockSpec((bn, D), lambda i: (i, 0)),
        out_shape=jax.ShapeDtypeStruct((N, D), jnp.float32),
    )(x, aggs, wn1, wn2, b_n)


_sc_mesh = plsc.VectorSubcoreMesh(core_axis_name="c", subcore_axis_name="s")


@functools.partial(
    pl.kernel,
    out_type=(
        jax.ShapeDtypeStruct((E, DOUT), jnp.float32),
        jax.ShapeDtypeStruct((NC, N, DOUT), jnp.float32),
    ),
    mesh=_sc_mesh,
    scratch_types=[
        pltpu.VMEM((C,), jnp.int32),
        pltpu.VMEM((C,), jnp.int32),
        pltpu.VMEM((C,), jnp.int32),
        pltpu.VMEM((C,), jnp.int32),
        pltpu.VMEM((C,), jnp.int32),
        pltpu.VMEM((C,), jnp.int32),
        pltpu.VMEM((TAILE,), jnp.int32),
        pltpu.VMEM((TAILE,), jnp.int32),
        pltpu.VMEM((C, DOUT), jnp.float32),
        pltpu.VMEM((C, DOUT), jnp.float32),
        pltpu.VMEM((C, DOUT), jnp.float32),
        pltpu.VMEM((C, DOUT), jnp.float32),
        pltpu.VMEM((C, DOUT), jnp.float32),
        pltpu.VMEM((C, DOUT), jnp.float32),
        pltpu.VMEM_SHARED((N, DOUT), jnp.float32),
        pltpu.SemaphoreType.DMA,
        pltpu.SemaphoreType.DMA,
        pltpu.SemaphoreType.DMA,
        pltpu.SemaphoreType.DMA,
        pltpu.SemaphoreType.DMA,
        pltpu.SemaphoreType.DMA,
        pltpu.SemaphoreType.DMA,
        pltpu.SemaphoreType.DMA,
        pltpu.SemaphoreType.DMA,
        pltpu.SemaphoreType.DMA,
        pltpu.SemaphoreType.DMA,
        pltpu.SemaphoreType.DMA,
    ],
)
def _edge_kernel(src_hbm, dst_hbm, psrc_hbm, pdst_hbm, patt_hbm,
                 eout_hbm, agg_hbm,
                 idx_s0, idx_d0, idx_s1, idx_d1, sidx0, sidx1,
                 idx_st, idx_dt,
                 a0, b0, c0, a1, b1, c1, agg_sh,
                 ga_sem0, gb_sem0, pc_sem0, ga_sem1, gb_sem1, pc_sem1,
                 eo_sem0, sc_sem0, eo_sem1, sc_sem1, ix_sem0, ix_sem1):
    cid = lax.axis_index("c")
    sid = lax.axis_index("s")
    wid = sid * NC + cid
    base_w = wid * EPW

    sets = (
        dict(idx_s=idx_s0, idx_d=idx_d0, a=a0, b=b0, c=c0, sidx=sidx0,
             ga=ga_sem0, gb=gb_sem0, pc=pc_sem0, eo=eo_sem0, sc=sc_sem0,
             ix=ix_sem0),
        dict(idx_s=idx_s1, idx_d=idx_d1, a=a1, b=b1, c=c1, sidx=sidx1,
             ga=ga_sem1, gb=gb_sem1, pc=pc_sem1, eo=eo_sem1, sc=sc_sem1,
             ix=ix_sem1),
    )

    def idx_descs(base, s):
        t = sets[s]
        return (pltpu.make_async_copy(src_hbm.at[pl.ds(base, C)],
                                      t["idx_s"], t["ix"]),
                pltpu.make_async_copy(dst_hbm.at[pl.ds(base, C)],
                                      t["idx_d"], t["ix"]))

    def in_descs(base, s):
        t = sets[s]
        return (pltpu.make_async_copy(psrc_hbm.at[t["idx_s"]],
                                      t["a"], t["ga"]),
                pltpu.make_async_copy(pdst_hbm.at[t["idx_d"]],
                                      t["b"], t["gb"]),
                pltpu.make_async_copy(patt_hbm.at[pl.ds(base, C)],
                                      t["c"], t["pc"]))

    def out_descs(base, s):
        t = sets[s]
        return (pltpu.make_async_copy(t["a"],
                                      eout_hbm.at[pl.ds(base, C)],
                                      t["eo"]),
                pltpu.make_async_copy(t["a"],
                                      agg_sh.at[t["sidx"]], t["sc"]))

    def snap_idx(s):
        t = sets[s]
        for g in range(C // L):
            sl = pl.ds(g * L, L)
            t["sidx"][sl] = t["idx_d"][sl]

    def fire_out(base, s):
        d = out_descs(base, s)
        d[0].start()
        d[1].start(add=True)

    def _compute(a, b, c, nrows):
        def _row(i, rcarry):
            for u in range(2):
                for g in range(DOUT // L):
                    sl = pl.ds(g * L, L)
                    r = 2 * i + u
                    a[r, sl] = jnp.maximum(
                        a[r, sl] + b[r, sl] + c[r, sl], 0.0)
            return rcarry

        lax.fori_loop(0, nrows // 2, _row, 0)

    for d in idx_descs(base_w, 0):
        d.start()

    def _zrow(i, carry):
        for g in range(DOUT // L):
            c1[i, pl.ds(g * L, L)] = jnp.zeros((L,), jnp.float32)
        return carry

    lax.fori_loop(0, C, _zrow, 0)
    for d in idx_descs(base_w, 0):
        d.wait()
    for d in in_descs(base_w, 0):
        d.start()
    for d in idx_descs(base_w + C, 1):
        d.start()

    nblk_me = jnp.where(sid < NBLK - 9 * NS, 10, 9)

    def _zblk(k, carry):
        pltpu.sync_copy(c1, agg_sh.at[pl.ds((sid + NS * k) * C, C)])
        return carry

    lax.fori_loop(0, nblk_me, _zblk, 0)

    @pl.when(sid == NS - 1)
    def _zero_tail():
        pltpu.sync_copy(c1.at[pl.ds(0, AGG_TAILR)],
                        agg_sh.at[pl.ds(AGG_TAIL0, AGG_TAILR)])

    plsc.subcore_barrier()

    def _pair(p, carry):
        base_a = base_w + 2 * p * C

        @pl.when(p > 0)
        def _():
            for d in out_descs(base_a - C, 1):
                d.wait()

        for d in idx_descs(base_a + C, 1):
            d.wait()
        for d in in_descs(base_a + C, 1):
            d.start()
        for d in in_descs(base_a, 0):
            d.wait()
        snap_idx(0)

        @pl.when(p < PAIRS - 1)
        def _():
            for d in idx_descs(base_a + 2 * C, 0):
                d.start()

        _compute(a0, b0, c0, C)
        fire_out(base_a, 0)

        base_b = base_a + C
        for d in out_descs(base_a, 0):
            d.wait()

        @pl.when(p < PAIRS - 1)
        def _():
            for d in idx_descs(base_b + C, 0):
                d.wait()
            for d in in_descs(base_b + C, 0):
                d.start()

        for d in in_descs(base_b, 1):
            d.wait()
        snap_idx(1)

        @pl.when(p < PAIRS - 1)
        def _():
            for d in idx_descs(base_b + 2 * C, 1):
                d.start()

        _compute(a1, b1, c1, C)
        fire_out(base_b, 1)
        return carry

    lax.fori_loop(0, PAIRS, _pair, 0)
    for d in out_descs(base_w + (NCH - 1) * C, 1):
        d.wait()

    tb = base_w + TBASE
    pltpu.sync_copy(src_hbm.at[pl.ds(tb, TAILE)], idx_st)
    pltpu.sync_copy(dst_hbm.at[pl.ds(tb, TAILE)], idx_dt)
    cp1 = pltpu.async_copy(psrc_hbm.at[idx_st], a0.at[pl.ds(0, TAILE)],
                           ga_sem0)
    cp2 = pltpu.async_copy(pdst_hbm.at[idx_dt], b0.at[pl.ds(0, TAILE)],
                           gb_sem0)
    cp3 = pltpu.async_copy(patt_hbm.at[pl.ds(tb, TAILE)],
                           c0.at[pl.ds(0, TAILE)], pc_sem0)
    cp1.wait()
    cp2.wait()
    cp3.wait()
    _compute(a0, b0, c0, TAILE)
    tr = a0.at[pl.ds(0, TAILE)]
    pltpu.sync_copy(tr, eout_hbm.at[pl.ds(tb, TAILE)])
    pltpu.sync_copy(tr, agg_sh.at[idx_dt], add=True)

    plsc.subcore_barrier()

    def _dblk(k, carry):
        off = (sid + NS * k) * C
        pltpu.sync_copy(agg_sh.at[pl.ds(off, C)], a0)
        pltpu.sync_copy(a0, agg_hbm.at[cid, pl.ds(off, C)])
        return carry

    lax.fori_loop(0, nblk_me, _dblk, 0)

    @pl.when(sid == NS - 1)
    def _drain_tail():
        pltpu.sync_copy(agg_sh.at[pl.ds(AGG_TAIL0, AGG_TAILR)],
                        a0.at[pl.ds(0, AGG_TAILR)])
        pltpu.sync_copy(a0.at[pl.ds(0, AGG_TAILR)],
                        agg_hbm.at[cid, pl.ds(AGG_TAIL0, AGG_TAILR)])


@jax.jit
def kernel(x, edge_index, edge_attr, W_e, b_e, W_n, b_n):
    src = edge_index[0]
    dst = edge_index[1]
    psrc, pdst = _proj(x, W_e[:D], W_e[D:2 * D])
    patt = _patt(edge_attr.T, W_e[2 * D:], b_e.reshape(1, DOUT))
    edge_out, aggs = _edge_kernel(src, dst, psrc, pdst, patt)
    x_out = _node(x, aggs, W_n[:D], W_n[D:], b_n.reshape(1, D))
    return (x_out, edge_out)


# speedup vs baseline: 1.3908x; 1.0064x over previous
import functools

import jax
import jax.numpy as jnp
from jax import lax
from jax.experimental import pallas as pl
from jax.experimental.pallas import tpu as pltpu
from jax.experimental.pallas import tpu_sc as plsc

N = 10000
E = 320000
D = 128
DE = 16
DOUT = 128

NC = 2
NS = 16
L = 16
NW = NC * NS
EPW = E // NW
C = 64
NCH = 156
PAIRS = NCH // 2
TAILE = EPW - NCH * C
TBASE = NCH * C
NBLK = N // C
AGG_TAIL0 = NBLK * C
AGG_TAILR = N - AGG_TAIL0


def _proj_body(x_ref, w1_ref, w2_ref, o1_ref, o2_ref):
    xb = x_ref[...]
    o1_ref[...] = jnp.dot(xb, w1_ref[...], preferred_element_type=jnp.float32)
    o2_ref[...] = jnp.dot(xb, w2_ref[...], preferred_element_type=jnp.float32)


def _proj(x, w1, w2):
    bn = 5000
    grid = N // bn
    return pl.pallas_call(
        _proj_body,
        grid=(grid,),
        in_specs=[
            pl.BlockSpec((bn, D), lambda i: (i, 0)),
            pl.BlockSpec((D, D), lambda i: (0, 0)),
            pl.BlockSpec((D, D), lambda i: (0, 0)),
        ],
        out_specs=[
            pl.BlockSpec((bn, D), lambda i: (i, 0)),
            pl.BlockSpec((bn, D), lambda i: (i, 0)),
        ],
        out_shape=[
            jax.ShapeDtypeStruct((N, D), jnp.float32),
            jax.ShapeDtypeStruct((N, D), jnp.float32),
        ],
    )(x, w1, w2)


def _patt_body(at_ref, w_ref, b_ref, o_ref):
    o_ref[...] = lax.dot_general(
        at_ref[...], w_ref[...],
        dimension_numbers=(((0,), (0,)), ((), ())),
        preferred_element_type=jnp.float32) + b_ref[...]


def _patt(edge_attr_t, w3, b_e):
    be = 32000
    grid = E // be
    return pl.pallas_call(
        _patt_body,
        grid=(grid,),
        in_specs=[
            pl.BlockSpec((DE, be), lambda i: (0, i)),
            pl.BlockSpec((DE, DOUT), lambda i: (0, 0)),
            pl.BlockSpec((1, DOUT), lambda i: (0, 0)),
        ],
        out_specs=pl.BlockSpec((be, DOUT), lambda i: (i, 0)),
        out_shape=jax.ShapeDtypeStruct((E, DOUT), jnp.float32),
    )(edge_attr_t, w3, b_e)


def _node_body(x_ref, a_ref, w1_ref, w2_ref, b_ref, o_ref):
    acc = jnp.dot(x_ref[...], w1_ref[...], preferred_element_type=jnp.float32)
    acc += jnp.dot(a_ref[0] + a_ref[1], w2_ref[...],
                   preferred_element_type=jnp.float32)
    o_ref[...] = jnp.maximum(acc + b_ref[...], 0.0)


def _node(x, aggs, wn1, wn2, b_n):
    bn = 2000
    grid = N // bn
    return pl.pallas_call(
        _node_body,
        grid=(grid,),
        in_specs=[
            pl.BlockSpec((bn, D), lambda i: (i, 0)),
            pl.BlockSpec((NC, bn, DOUT), lambda i: (0, i, 0)),
            pl.BlockSpec((D, D), lambda i: (0, 0)),
            pl.BlockSpec((DOUT, D), lambda i: (0, 0)),
            pl.BlockSpec((1, D), lambda i: (0, 0)),
        ],
        out_specs=pl.BlockSpec((bn, D), lambda i: (i, 0)),
        out_shape=jax.ShapeDtypeStruct((N, D), jnp.float32),
    )(x, aggs, wn1, wn2, b_n)


_sc_mesh = plsc.VectorSubcoreMesh(core_axis_name="c", subcore_axis_name="s")


@functools.partial(
    pl.kernel,
    out_type=(
        jax.ShapeDtypeStruct((E, DOUT), jnp.float32),
        jax.ShapeDtypeStruct((NC, N, DOUT), jnp.float32),
    ),
    mesh=_sc_mesh,
    scratch_types=[
        pltpu.VMEM((C,), jnp.int32),
        pltpu.VMEM((C,), jnp.int32),
        pltpu.VMEM((C,), jnp.int32),
        pltpu.VMEM((C,), jnp.int32),
        pltpu.VMEM((C,), jnp.int32),
        pltpu.VMEM((C,), jnp.int32),
        pltpu.VMEM((TAILE,), jnp.int32),
        pltpu.VMEM((TAILE,), jnp.int32),
        pltpu.VMEM((C, DOUT), jnp.float32),
        pltpu.VMEM((C, DOUT), jnp.float32),
        pltpu.VMEM((C, DOUT), jnp.float32),
        pltpu.VMEM((C, DOUT), jnp.float32),
        pltpu.VMEM((C, DOUT), jnp.float32),
        pltpu.VMEM((C, DOUT), jnp.float32),
        pltpu.VMEM_SHARED((N, DOUT), jnp.float32),
        pltpu.SemaphoreType.DMA,
        pltpu.SemaphoreType.DMA,
        pltpu.SemaphoreType.DMA,
        pltpu.SemaphoreType.DMA,
        pltpu.SemaphoreType.DMA,
        pltpu.SemaphoreType.DMA,
        pltpu.SemaphoreType.DMA,
        pltpu.SemaphoreType.DMA,
        pltpu.SemaphoreType.DMA,
        pltpu.SemaphoreType.DMA,
        pltpu.SemaphoreType.DMA,
        pltpu.SemaphoreType.DMA,
    ],
)
def _edge_kernel(src_hbm, dst_hbm, psrc_hbm, pdst_hbm, patt_hbm,
                 eout_hbm, agg_hbm,
                 idx_s0, idx_d0, idx_s1, idx_d1, sidx0, sidx1,
                 idx_st, idx_dt,
                 a0, b0, c0, a1, b1, c1, agg_sh,
                 ga_sem0, gb_sem0, pc_sem0, ga_sem1, gb_sem1, pc_sem1,
                 eo_sem0, sc_sem0, eo_sem1, sc_sem1, ix_sem0, ix_sem1):
    cid = lax.axis_index("c")
    sid = lax.axis_index("s")
    wid = sid * NC + cid
    base_w = wid * EPW

    sets = (
        dict(idx_s=idx_s0, idx_d=idx_d0, a=a0, b=b0, c=c0, sidx=sidx0,
             ga=ga_sem0, gb=gb_sem0, pc=pc_sem0, eo=eo_sem0, sc=sc_sem0,
             ix=ix_sem0),
        dict(idx_s=idx_s1, idx_d=idx_d1, a=a1, b=b1, c=c1, sidx=sidx1,
             ga=ga_sem1, gb=gb_sem1, pc=pc_sem1, eo=eo_sem1, sc=sc_sem1,
             ix=ix_sem1),
    )

    def idx_descs(base, s):
        t = sets[s]
        return (pltpu.make_async_copy(src_hbm.at[pl.ds(base, C)],
                                      t["idx_s"], t["ix"]),
                pltpu.make_async_copy(dst_hbm.at[pl.ds(base, C)],
                                      t["idx_d"], t["ix"]))

    def in_descs(base, s):
        t = sets[s]
        return (pltpu.make_async_copy(psrc_hbm.at[t["idx_s"]],
                                      t["a"], t["ga"]),
                pltpu.make_async_copy(pdst_hbm.at[t["idx_d"]],
                                      t["b"], t["gb"]),
                pltpu.make_async_copy(patt_hbm.at[pl.ds(base, C)],
                                      t["c"], t["pc"]))

    def out_descs(base, s):
        t = sets[s]
        return (pltpu.make_async_copy(t["a"],
                                      eout_hbm.at[pl.ds(base, C)],
                                      t["eo"]),
                pltpu.make_async_copy(t["a"],
                                      agg_sh.at[t["sidx"]], t["sc"]))

    def snap_idx(s):
        t = sets[s]
        for g in range(C // L):
            sl = pl.ds(g * L, L)
            t["sidx"][sl] = t["idx_d"][sl]

    def fire_out(base, s):
        d = out_descs(base, s)
        d[0].start()
        d[1].start(add=True)

    def _compute(a, b, c, nrows):
        def _row(i, rcarry):
            for u in range(2):
                for g in range(DOUT // L):
                    sl = pl.ds(g * L, L)
                    r = 2 * i + u
                    a[r, sl] = jnp.maximum(
                        a[r, sl] + b[r, sl] + c[r, sl], 0.0)
            return rcarry

        lax.fori_loop(0, nrows // 2, _row, 0)

    for d in idx_descs(base_w, 0):
        d.start()

    def _zrow(i, carry):
        for g in range(DOUT // L):
            c1[i, pl.ds(g * L, L)] = jnp.zeros((L,), jnp.float32)
        return carry

    lax.fori_loop(0, C, _zrow, 0)
    for d in idx_descs(base_w, 0):
        d.wait()
    for d in in_descs(base_w, 0):
        d.start()
    for d in idx_descs(base_w + C, 1):
        d.start()

    nblk_me = jnp.where(sid < NBLK - 9 * NS, 10, 9)

    def _zblk(k, carry):
        pltpu.sync_copy(c1, agg_sh.at[pl.ds((sid + NS * k) * C, C)])
        return carry

    lax.fori_loop(0, nblk_me, _zblk, 0)

    @pl.when(sid == NS - 1)
    def _zero_tail():
        pltpu.sync_copy(c1.at[pl.ds(0, AGG_TAILR)],
                        agg_sh.at[pl.ds(AGG_TAIL0, AGG_TAILR)])

    plsc.subcore_barrier()

    def _pair(p, carry):
        base_a = base_w + 2 * p * C

        @pl.when(p > 0)
        def _():
            for d in out_descs(base_a - C, 1):
                d.wait()

        for d in idx_descs(base_a + C, 1):
            d.wait()
        for d in in_descs(base_a + C, 1):
            d.start()
        for d in in_descs(base_a, 0):
            d.wait()
        snap_idx(0)

        @pl.when(p < PAIRS - 1)
        def _():
            for d in idx_descs(base_a + 2 * C, 0):
                d.start()

        _compute(a0, b0, c0, C)
        fire_out(base_a, 0)

        base_b = base_a + C
        for d in out_descs(base_a, 0):
            d.wait()

        @pl.when(p < PAIRS - 1)
        def _():
            for d in idx_descs(base_b + C, 0):
                d.wait()
            for d in in_descs(base_b + C, 0):
                d.start()

        for d in in_descs(base_b, 1):
            d.wait()
        snap_idx(1)

        @pl.when(p < PAIRS - 1)
        def _():
            for d in idx_descs(base_b + 2 * C, 1):
                d.start()

        _compute(a1, b1, c1, C)
        fire_out(base_b, 1)
        return carry

    lax.fori_loop(0, PAIRS, _pair, 0)
    for d in out_descs(base_w + (NCH - 1) * C, 1):
        d.wait()

    tb = base_w + TBASE
    pltpu.sync_copy(src_hbm.at[pl.ds(tb, TAILE)], idx_st)
    pltpu.sync_copy(dst_hbm.at[pl.ds(tb, TAILE)], idx_dt)
    cp1 = pltpu.async_copy(psrc_hbm.at[idx_st], a0.at[pl.ds(0, TAILE)],
                           ga_sem0)
    cp2 = pltpu.async_copy(pdst_hbm.at[idx_dt], b0.at[pl.ds(0, TAILE)],
                           gb_sem0)
    cp3 = pltpu.async_copy(patt_hbm.at[pl.ds(tb, TAILE)],
                           c0.at[pl.ds(0, TAILE)], pc_sem0)
    cp1.wait()
    cp2.wait()
    cp3.wait()
    _compute(a0, b0, c0, TAILE)
    tr = a0.at[pl.ds(0, TAILE)]
    pltpu.sync_copy(tr, eout_hbm.at[pl.ds(tb, TAILE)])
    pltpu.sync_copy(tr, agg_sh.at[idx_dt], add=True)

    plsc.subcore_barrier()

    def _dblk(k, carry):
        off = (sid + NS * k) * C
        pltpu.sync_copy(agg_sh.at[pl.ds(off, C)], a0)
        pltpu.sync_copy(a0, agg_hbm.at[cid, pl.ds(off, C)])
        return carry

    lax.fori_loop(0, nblk_me, _dblk, 0)

    @pl.when(sid == NS - 1)
    def _drain_tail():
        pltpu.sync_copy(agg_sh.at[pl.ds(AGG_TAIL0, AGG_TAILR)],
                        a0.at[pl.ds(0, AGG_TAILR)])
        pltpu.sync_copy(a0.at[pl.ds(0, AGG_TAILR)],
                        agg_hbm.at[cid, pl.ds(AGG_TAIL0, AGG_TAILR)])


@jax.jit
def kernel(x, edge_index, edge_attr, W_e, b_e, W_n, b_n):
    src = edge_index[0]
    dst = edge_index[1]
    psrc, pdst = _proj(x, W_e[:D], W_e[D:2 * D])
    patt = _patt(edge_attr.T, W_e[2 * D:], b_e.reshape(1, DOUT))
    edge_out, aggs = _edge_kernel(src, dst, psrc, pdst, patt)
    x_out = _node(x, aggs, W_n[:D], W_n[D:], b_n.reshape(1, D))
    return (x_out, edge_out)
